# baseline (speedup 1.0000x reference)
import functools

import jax
import jax.numpy as jnp
from jax import lax
from jax.experimental import pallas as pl
from jax.experimental.pallas import tpu as pltpu
from jax.experimental.pallas import tpu_sc as plsc

L = 1024
D = 64
NRES = 66
NATOM = 10
NC = 2
NS = 16
NW = NC * NS
ROWS_PER_W = L // NW
LANES = 16


def _table_body(res_ref, atom_ref, t_ref):
    t_ref[...] = res_ref[...][:, None, :] + atom_ref[...][None, :, :]


def _build_table(emb_res_W, emb_atom_W):
    t3 = pl.pallas_call(
        _table_body,
        out_shape=jax.ShapeDtypeStruct((NRES, NATOM, D), jnp.float32),
    )(emb_res_W, emb_atom_W)
    return t3.reshape(NRES * NATOM, D)


CHUNK = 512
NCHUNK = ROWS_PER_W * 2


def _sc_body(table_hbm, idx_hbm, msk_hbm, idxb_hbm, mskb_hbm, dist_hbm, out_hbm,
             tbl_v, idx_v, msk_v, idxb_v, mskb_v, dist_v, rows_v,
             osemA, osemB, dsemA, dsemB):
    wid = lax.axis_index("s") * NC + lax.axis_index("c")
    row0 = wid * ROWS_PER_W
    pltpu.sync_copy(table_hbm, tbl_v)
    pltpu.sync_copy(idx_hbm, idx_v)
    pltpu.sync_copy(msk_hbm, msk_v)
    pltpu.sync_copy(idxb_hbm.at[pl.ds(row0, ROWS_PER_W)], idxb_v)
    pltpu.sync_copy(mskb_hbm.at[pl.ds(row0, ROWS_PER_W)], mskb_v)
    lane = lax.iota(jnp.int32, LANES)

    def out_dst(c):
        return out_hbm.at[0, row0 + (c >> 1), pl.ds((c & 1) * (CHUNK // 2), CHUNK // 2)]

    def dist_src(c):
        return dist_hbm.at[row0 + (c >> 1), pl.ds((c & 1) * CHUNK, CHUNK)]

    def chunk_work(c, b, par):
        r = c >> 1
        colb = (c & 1) * CHUNK
        idx_i = idxb_v[r]
        msk_i = mskb_v[r]
        rows_b = rows_v.at[b]

        @plsc.parallel_loop(0, CHUNK // LANES, unroll=2)
        def group(s):
            col = colb + s * LANES
            idx_j = idx_v[pl.ds(col, LANES)]
            msk_j = msk_v[pl.ds(col, LANES)]
            d = dist_v[b, par, pl.ds(s * LANES, LANES)]
            res = jnp.clip(idx_j - idx_i, -32, 33) + 32
            res = jnp.where(msk_i + msk_j == 0, res, 65)
            dc = jnp.clip(d, 0.0, 9.0)
            tr = dc.astype(jnp.int32)
            ia = jnp.where(dc > tr.astype(jnp.float32), tr + 1, tr)
            ia = jnp.where(msk_i + msk_j == 2, ia, 9)
            cid = res * NATOM + ia
            src0 = cid * D
            for p in range(LANES):
                bc = jnp.max(jnp.where(lane == p, src0, 0))
                idx0 = bc + lane
                sp = s * LANES + p
                for t in range(D // LANES):
                    v = plsc.load_gather(tbl_v, [idx0 + t * LANES])
                    rows_b[sp >> 1, pl.ds((sp & 1) * D + t * LANES, LANES)] = v

    pltpu.sync_copy(dist_src(0), dist_v.at[0, 0])
    pltpu.sync_copy(dist_src(1), dist_v.at[1, 0])

    def step(i, carry):
        a, bb = 2 * i, 2 * i + 1
        par = i & 1

        @pl.when(i > 0)
        def _():
            pltpu.make_async_copy(dist_src(a), dist_v.at[0, par], dsemA).wait()
            pltpu.make_async_copy(dist_src(bb), dist_v.at[1, par], dsemB).wait()

        @pl.when(i < NCHUNK // 2 - 1)
        def _():
            pltpu.async_copy(dist_src(a + 2), dist_v.at[0, 1 - par], dsemA)
            pltpu.async_copy(dist_src(bb + 2), dist_v.at[1, 1 - par], dsemB)

        @pl.when(i > 0)
        def _():
            pltpu.make_async_copy(rows_v.at[0], out_dst(a - 2), osemA).wait()
            pltpu.make_async_copy(rows_v.at[1], out_dst(a - 1), osemB).wait()

        chunk_work(a, 0, par)
        pltpu.async_copy(rows_v.at[0], out_dst(a), osemA)
        chunk_work(bb, 1, par)
        pltpu.async_copy(rows_v.at[1], out_dst(bb), osemB)
        return carry

    lax.fori_loop(0, NCHUNK // 2, step, 0)
    pltpu.make_async_copy(rows_v.at[0], out_dst(NCHUNK - 2), osemA).wait()
    pltpu.make_async_copy(rows_v.at[1], out_dst(NCHUNK - 1), osemB).wait()


def kernel(seq, idx, bond_feats, dist_matrix, emb_res_W, emb_atom_W):
    del bond_feats
    table = _build_table(emb_res_W, emb_atom_W).reshape(-1)
    idx32 = idx[0].astype(jnp.int32)
    msk32 = (seq[0] >= 32).astype(jnp.int32)
    idxb = jnp.tile(idx32[:, None], (1, LANES))
    mskb = jnp.tile(msk32[:, None], (1, LANES))
    dist = dist_matrix[0]
    sc = pl.kernel(
        _sc_body,
        out_type=jax.ShapeDtypeStruct((1, L, L // 2, 2 * D), jnp.float32),
        mesh=plsc.VectorSubcoreMesh(core_axis_name="c", subcore_axis_name="s"),
        scratch_types=[
            pltpu.VMEM((NRES * NATOM * D,), jnp.float32),
            pltpu.VMEM((L,), jnp.int32),
            pltpu.VMEM((L,), jnp.int32),
            pltpu.VMEM((ROWS_PER_W, LANES), jnp.int32),
            pltpu.VMEM((ROWS_PER_W, LANES), jnp.int32),
            pltpu.VMEM((2, 2, CHUNK), jnp.float32),
            pltpu.VMEM((2, CHUNK // 2, 2 * D), jnp.float32),
            pltpu.SemaphoreType.DMA,
            pltpu.SemaphoreType.DMA,
            pltpu.SemaphoreType.DMA,
            pltpu.SemaphoreType.DMA,
        ],
        compiler_params=pltpu.CompilerParams(use_tc_tiling_on_sc=False,
                                             needs_layout_passes=False),
    )
    return sc(table, idx32, msk32, idxb, mskb, dist).reshape(1, L, L, D)


# speedup vs baseline: 1.4382x; 1.4382x over previous
import functools

import jax
import jax.numpy as jnp
from jax import lax
from jax.experimental import pallas as pl
from jax.experimental.pallas import tpu as pltpu
from jax.experimental.pallas import tpu_sc as plsc

L = 1024
D = 64
NRES = 66
NATOM = 10
NC = 2
NS = 16
NW = NC * NS
ROWS_PER_W = L // NW
LANES = 16


def _table_body(res_ref, atom_ref, t_ref):
    t_ref[...] = res_ref[...][:, None, :] + atom_ref[...][None, :, :]


def _build_table(emb_res_W, emb_atom_W):
    t3 = pl.pallas_call(
        _table_body,
        out_shape=jax.ShapeDtypeStruct((NRES, NATOM, D), jnp.float32),
    )(emb_res_W, emb_atom_W)
    return t3.reshape(NRES * NATOM, D)


CHUNK = 512
NCHUNK = ROWS_PER_W * 2


def _sc_body(table_hbm, idx_hbm, msk_hbm, idxb_hbm, mskb_hbm, dist_hbm, out_hbm,
             tbl_v, idx_v, msk_v, idxb_v, mskb_v, dist_v, rows_v,
             osemA, osemB, dsemA, dsemB):
    wid = lax.axis_index("s") * NC + lax.axis_index("c")
    row0 = wid * ROWS_PER_W
    pltpu.sync_copy(table_hbm, tbl_v)
    pltpu.sync_copy(idx_hbm, idx_v)
    pltpu.sync_copy(msk_hbm, msk_v)
    pltpu.sync_copy(idxb_hbm.at[pl.ds(row0, ROWS_PER_W)], idxb_v)
    pltpu.sync_copy(mskb_hbm.at[pl.ds(row0, ROWS_PER_W)], mskb_v)
    lane = lax.iota(jnp.int32, LANES)

    def out_dst(c):
        return out_hbm.at[0, row0 + (c >> 1), pl.ds((c & 1) * (CHUNK // 2), CHUNK // 2)]

    def dist_src(c):
        return dist_hbm.at[row0 + (c >> 1), pl.ds((c & 1) * CHUNK, CHUNK)]

    def chunk_work(c, b, par):
        r = c >> 1
        colb = (c & 1) * CHUNK
        idx_i = idxb_v[r]
        msk_i = mskb_v[r]
        rows_b = rows_v.at[b]

        @plsc.parallel_loop(0, CHUNK // LANES)
        def group(s):
            col = colb + s * LANES
            idx_j = idx_v[pl.ds(col, LANES)]
            msk_j = msk_v[pl.ds(col, LANES)]
            d = dist_v[b, par, pl.ds(s * LANES, LANES)]
            res = jnp.clip(idx_j - idx_i, -32, 33) + 32
            res = jnp.where(msk_i + msk_j == 0, res, 65)
            dc = jnp.clip(d, 0.0, 9.0)
            tr = dc.astype(jnp.int32)
            ia = jnp.where(dc > tr.astype(jnp.float32), tr + 1, tr)
            ia = jnp.where(msk_i + msk_j == 2, ia, 9)
            cid = res * NATOM + ia
            src0 = cid * D
            for p in range(LANES):
                bc = jnp.max(jnp.where(lane == p, src0, 0))
                idx0 = bc + lane
                sp = s * LANES + p
                for t in range(D // LANES):
                    v = plsc.load_gather(tbl_v, [idx0 + t * LANES])
                    rows_b[sp >> 1, pl.ds((sp & 1) * D + t * LANES, LANES)] = v

    pltpu.sync_copy(dist_src(0), dist_v.at[0, 0])
    pltpu.sync_copy(dist_src(1), dist_v.at[1, 0])

    def step(i, carry):
        a, bb = 2 * i, 2 * i + 1
        par = i & 1

        @pl.when(i > 0)
        def _():
            pltpu.make_async_copy(dist_src(a), dist_v.at[0, par], dsemA).wait()
            pltpu.make_async_copy(dist_src(bb), dist_v.at[1, par], dsemB).wait()

        @pl.when(i < NCHUNK // 2 - 1)
        def _():
            pltpu.async_copy(dist_src(a + 2), dist_v.at[0, 1 - par], dsemA)
            pltpu.async_copy(dist_src(bb + 2), dist_v.at[1, 1 - par], dsemB)

        @pl.when(i > 0)
        def _():
            pltpu.make_async_copy(rows_v.at[0], out_dst(a - 2), osemA).wait()
            pltpu.make_async_copy(rows_v.at[1], out_dst(a - 1), osemB).wait()

        chunk_work(a, 0, par)
        pltpu.async_copy(rows_v.at[0], out_dst(a), osemA)
        chunk_work(bb, 1, par)
        pltpu.async_copy(rows_v.at[1], out_dst(bb), osemB)
        return carry

    lax.fori_loop(0, NCHUNK // 2, step, 0)
    pltpu.make_async_copy(rows_v.at[0], out_dst(NCHUNK - 2), osemA).wait()
    pltpu.make_async_copy(rows_v.at[1], out_dst(NCHUNK - 1), osemB).wait()


def kernel(seq, idx, bond_feats, dist_matrix, emb_res_W, emb_atom_W):
    del bond_feats
    table = _build_table(emb_res_W, emb_atom_W).reshape(-1)
    idx32 = idx[0].astype(jnp.int32)
    msk32 = (seq[0] >= 32).astype(jnp.int32)
    idxb = jnp.tile(idx32[:, None], (1, LANES))
    mskb = jnp.tile(msk32[:, None], (1, LANES))
    dist = dist_matrix[0]
    sc = pl.kernel(
        _sc_body,
        out_type=jax.ShapeDtypeStruct((1, L, L // 2, 2 * D), jnp.float32),
        mesh=plsc.VectorSubcoreMesh(core_axis_name="c", subcore_axis_name="s"),
        scratch_types=[
            pltpu.VMEM((NRES * NATOM * D,), jnp.float32),
            pltpu.VMEM((L,), jnp.int32),
            pltpu.VMEM((L,), jnp.int32),
            pltpu.VMEM((ROWS_PER_W, LANES), jnp.int32),
            pltpu.VMEM((ROWS_PER_W, LANES), jnp.int32),
            pltpu.VMEM((2, 2, CHUNK), jnp.float32),
            pltpu.VMEM((2, CHUNK // 2, 2 * D), jnp.float32),
            pltpu.SemaphoreType.DMA,
            pltpu.SemaphoreType.DMA,
            pltpu.SemaphoreType.DMA,
            pltpu.SemaphoreType.DMA,
        ],
        compiler_params=pltpu.CompilerParams(use_tc_tiling_on_sc=False,
                                             needs_layout_passes=False),
    )
    return sc(table, idx32, msk32, idxb, mskb, dist).reshape(1, L, L, D)
